# baseline (speedup 1.0000x reference)
import functools

import jax
import jax.numpy as jnp
from jax import lax
from jax.experimental import pallas as pl
from jax.experimental.pallas import tpu as pltpu
from jax.experimental.pallas import tpu_sc as plsc

WORLD = 17
HALF = 32
NB = 4096
NT = 200
NW = 32
NTT = NT // 8
TAB = (WORLD * WORLD + 4) * HALF

_mesh = plsc.VectorSubcoreMesh(core_axis_name="c", subcore_axis_name="s")


@functools.partial(
    pl.kernel,
    out_type=jax.ShapeDtypeStruct((NT, 8, NW, 8 * 128), jnp.float32),
    mesh=_mesh,
    compiler_params=pltpu.CompilerParams(
        needs_layout_passes=False, use_tc_tiling_on_sc=False
    ),
    scratch_types=[
        pltpu.VMEM((TAB,), jnp.float32),
        pltpu.VMEM((3, 8, 128), jnp.int32),
        pltpu.VMEM((2, 4, 8, 8 * 128), jnp.float32),
        pltpu.SemaphoreType.DMA,
        pltpu.SemaphoreType.DMA,
    ],
)
def _sc_lookup(x_hbm, table_hbm, out_hbm, tab, xv, ov, sem0, sem1):
    bt = lax.axis_index("s") * 2 + lax.axis_index("c")
    pltpu.sync_copy(table_hbm, tab)
    sems = (sem0, sem1)

    def superblock(tt, carry):
        t0 = tt * 8
        pltpu.sync_copy(
            x_hbm.at[:, pl.ds(t0, 8), pl.ds(bt * 128, 128)], xv
        )
        for h in range(2):
            dst = out_hbm.at[pl.ds(t0 + h * 4, 4), :, bt]

            @pl.when(tt != 0)
            def _wait_prev():
                pltpu.make_async_copy(ov.at[h], dst, sems[h]).wait()

            @plsc.parallel_loop(0, 32, unroll=4)
            def _grp(q):
                t8 = h * 4 + q // 8
                g = q % 8
                rv = xv[0, t8, pl.ds(g * 16, 16)]
                cv = xv[1, t8, pl.ds(g * 16, 16)]
                dv = xv[2, t8, pl.ds(g * 16, 16)]
                gv = (rv * WORLD + cv) * HALF
                hv = (dv + WORLD * WORLD) * HALF
                for f in range(2 * HALF):
                    idx = gv + f if f < HALF else hv + (f - HALF)
                    vals = plsc.load_gather(tab, [idx])
                    ov[h, q // 8, f // 8, pl.ds((f % 8) * 128 + g * 16, 16)] = (
                        vals
                    )

            pltpu.async_copy(ov.at[h], dst, sems[h])
        return carry

    lax.fori_loop(0, NTT, superblock, 0)
    for h in range(2):
        pltpu.make_async_copy(
            ov.at[h],
            out_hbm.at[pl.ds((NTT - 1) * 8 + h * 4, 4), :, bt],
            sems[h],
        ).wait()


def kernel(x, grid_table, dir_table):
    table = jnp.concatenate([grid_table, dir_table], axis=0)
    xt = x.astype(jnp.int32).transpose(2, 1, 0)
    out = _sc_lookup(xt, table.reshape(-1))
    y = out.reshape(NT, 8, NW, 8, 128)
    y = y.transpose(2, 4, 0, 1, 3)
    return y.reshape(NB, NT, 2 * HALF)


# speedup vs baseline: 1.0624x; 1.0624x over previous
import functools

import jax
import jax.numpy as jnp
from jax import lax
from jax.experimental import pallas as pl
from jax.experimental.pallas import tpu as pltpu
from jax.experimental.pallas import tpu_sc as plsc

WORLD = 17
HALF = 32
NB = 4096
NT = 200
NW = 32
NTT = NT // 8
TAB = (WORLD * WORLD + 4) * HALF

_mesh = plsc.VectorSubcoreMesh(core_axis_name="c", subcore_axis_name="s")


@functools.partial(
    pl.kernel,
    out_type=jax.ShapeDtypeStruct((NT, 8, NW, 8 * 128), jnp.float32),
    mesh=_mesh,
    compiler_params=pltpu.CompilerParams(
        needs_layout_passes=False, use_tc_tiling_on_sc=False
    ),
    scratch_types=[
        pltpu.VMEM((TAB,), jnp.float32),
        pltpu.VMEM((3, 8, 128), jnp.int32),
        pltpu.VMEM((2, 4, 8, 8 * 128), jnp.float32),
        pltpu.SemaphoreType.DMA,
        pltpu.SemaphoreType.DMA,
    ],
)
def _sc_lookup(x_hbm, table_hbm, out_hbm, tab, xv, ov, sem0, sem1):
    bt = lax.axis_index("s") * 2 + lax.axis_index("c")
    pltpu.sync_copy(table_hbm, tab)
    sems = (sem0, sem1)

    def superblock(tt, carry):
        t0 = tt * 8
        pltpu.sync_copy(
            x_hbm.at[:, pl.ds(t0, 8), pl.ds(bt * 128, 128)], xv
        )
        for h in range(2):
            dst = out_hbm.at[pl.ds(t0 + h * 4, 4), :, bt]

            @pl.when(tt != 0)
            def _wait_prev():
                pltpu.make_async_copy(ov.at[h], dst, sems[h]).wait()

            @plsc.parallel_loop(0, 32, unroll=2)
            def _grp(q):
                t8 = h * 4 + q // 8
                g = q % 8
                rv = xv[0, t8, pl.ds(g * 16, 16)]
                cv = xv[1, t8, pl.ds(g * 16, 16)]
                dv = xv[2, t8, pl.ds(g * 16, 16)]
                gv = (rv * WORLD + cv) * HALF
                hv = (dv + WORLD * WORLD) * HALF
                for f in range(2 * HALF):
                    idx = gv + f if f < HALF else hv + (f - HALF)
                    vals = plsc.load_gather(tab, [idx])
                    ov[h, q // 8, f // 8, pl.ds((f % 8) * 128 + g * 16, 16)] = (
                        vals
                    )

            pltpu.async_copy(ov.at[h], dst, sems[h])
        return carry

    lax.fori_loop(0, NTT, superblock, 0)
    for h in range(2):
        pltpu.make_async_copy(
            ov.at[h],
            out_hbm.at[pl.ds((NTT - 1) * 8 + h * 4, 4), :, bt],
            sems[h],
        ).wait()


def kernel(x, grid_table, dir_table):
    table = jnp.concatenate([grid_table, dir_table], axis=0)
    xt = x.astype(jnp.int32).transpose(2, 1, 0)
    out = _sc_lookup(xt, table.reshape(-1))
    y = out.reshape(NT, 8, NW, 8, 128)
    y = y.transpose(2, 4, 0, 1, 3)
    return y.reshape(NB, NT, 2 * HALF)
